# baseline (speedup 1.0000x reference)
import functools

import jax
import jax.numpy as jnp
import numpy as np
from jax import lax
from jax.experimental import pallas as pl
from jax.experimental.pallas import tpu as pltpu
from jax.experimental.pallas import tpu_sc as plsc

_SC_CHUNK = 128


def _gather_rows_sc(ids_flat, table):
    (t_total,) = ids_flat.shape
    _, d = table.shape
    nw = 32
    per_w = t_total // nw
    n_chunks = per_w // _SC_CHUNK
    mesh = plsc.VectorSubcoreMesh(core_axis_name="c", subcore_axis_name="s")

    @functools.partial(
        pl.kernel,
        mesh=mesh,
        out_type=jax.ShapeDtypeStruct((t_total, d), jnp.float32),
        scratch_types=[
            pltpu.VMEM((_SC_CHUNK,), jnp.int32),
            pltpu.VMEM((_SC_CHUNK, d), jnp.float32),
            pltpu.SemaphoreType.DMA,
        ],
        compiler_params=pltpu.CompilerParams(use_tc_tiling_on_sc=False),
    )
    def gather_kernel(table_hbm, idx_hbm, out_hbm, idx_v, rows_v, sem):
        wid = lax.axis_index("s") * 2 + lax.axis_index("c")
        base = wid * per_w

        def body(i, carry):
            off = pl.multiple_of(base + i * _SC_CHUNK, _SC_CHUNK)
            pltpu.sync_copy(idx_hbm.at[pl.ds(off, _SC_CHUNK)], idx_v)
            pltpu.async_copy(table_hbm.at[idx_v], rows_v, sem).wait()
            pltpu.sync_copy(rows_v, out_hbm.at[pl.ds(off, _SC_CHUNK)])
            return carry

        lax.fori_loop(0, n_chunks, body, 0)

    return gather_kernel(table, ids_flat)


def _rotate_tc(x2, a, block_t):
    t_total, lanes = x2.shape
    d = lanes // 2

    e_np = np.zeros((d, lanes), np.float32)
    e_np[np.arange(d), 2 * np.arange(d)] = 1.0
    e_np[np.arange(d), 2 * np.arange(d) + 1] = 1.0
    p_np = np.zeros((lanes, lanes), np.float32)
    p_np[2 * np.arange(d) + 1, 2 * np.arange(d)] = 1.0
    p_np[2 * np.arange(d), 2 * np.arange(d) + 1] = 1.0
    e_mat = jnp.asarray(e_np)
    p_mat = jnp.asarray(p_np)

    def body(x_ref, a_ref, e_ref, p_ref, o_ref):
        v = x_ref[...]
        av = a_ref[...]
        asq = av * av
        recip = 1.0 / (1.0 + asq)
        cos_l = (1.0 - asq) * recip
        sin_l = (2.0 * av) * recip
        cos128 = lax.dot(cos_l, e_ref[...], precision=lax.Precision.HIGHEST)
        sin128 = lax.dot(sin_l, e_ref[...], precision=lax.Precision.HIGHEST)
        swapped = lax.dot(v, p_ref[...], precision=lax.Precision.HIGHEST)
        lane = lax.broadcasted_iota(jnp.int32, v.shape, 1)
        sgn = jnp.where(lane % 2 == 0, -1.0, 1.0)
        o_ref[...] = v * cos128 + swapped * (sin128 * sgn)

    return pl.pallas_call(
        body,
        grid=(t_total // block_t,),
        in_specs=[
            pl.BlockSpec((block_t, lanes), lambda i: (i, 0)),
            pl.BlockSpec((block_t, d), lambda i: (i, 0)),
            pl.BlockSpec((d, lanes), lambda i: (0, 0)),
            pl.BlockSpec((lanes, lanes), lambda i: (0, 0)),
        ],
        out_specs=pl.BlockSpec((block_t, lanes), lambda i: (i, 0)),
        out_shape=jax.ShapeDtypeStruct((t_total, lanes), jnp.float32),
        compiler_params=pltpu.CompilerParams(
            dimension_semantics=("arbitrary",),
        ),
    )(x2, a, e_mat, p_mat)


def kernel(x, affix_ids, rotation_params):
    b, s, d, _ = x.shape
    ids_flat = affix_ids.reshape(-1).astype(jnp.int32)
    x2 = x.reshape(b * s, d * 2)
    a = _gather_rows_sc(ids_flat, rotation_params)
    out2 = _rotate_tc(x2, a, block_t=512)
    return out2.reshape(b, s, d, 2)


# speedup vs baseline: 2.1994x; 2.1994x over previous
import functools

import jax
import jax.numpy as jnp
from jax import lax
from jax.experimental import pallas as pl
from jax.experimental.pallas import tpu as pltpu
from jax.experimental.pallas import tpu_sc as plsc

_LANES = 128
_NW = 32
_VOCAB_DIM = 64


def _gather_transpose_sc(ids_lin, table):
    (t_total,) = ids_lin.shape
    _, d = table.shape
    n_units = t_total // _LANES
    units_per_w = n_units // _NW
    mesh = plsc.VectorSubcoreMesh(core_axis_name="c", subcore_axis_name="s")

    @functools.partial(
        pl.kernel,
        mesh=mesh,
        out_type=jax.ShapeDtypeStruct((t_total // (8 * _LANES), d, 8, _LANES),
                                      jnp.float32),
        scratch_types=[
            pltpu.VMEM((_LANES,), jnp.int32),
            pltpu.VMEM((_LANES, d), jnp.float32),
            pltpu.VMEM((d, _LANES), jnp.float32),
            pltpu.SemaphoreType.DMA,
        ],
        compiler_params=pltpu.CompilerParams(
            use_tc_tiling_on_sc=False,
            needs_layout_passes=False,
        ),
    )
    def gather_kernel(table_hbm, idx_hbm, out_hbm, idx_v, rows_v, tr_v, sem):
        wid = lax.axis_index("s") * 2 + lax.axis_index("c")

        def unit_body(i, carry):
            u = wid * units_per_w + i
            s = u // 8
            bt = u % 8
            off = pl.multiple_of(u * _LANES, _LANES)
            pltpu.sync_copy(idx_hbm.at[pl.ds(off, _LANES)], idx_v)
            pltpu.async_copy(table_hbm.at[idx_v], rows_v, sem).wait()

            def tr_body(dd, c2):
                for lg in range(_LANES // 16):
                    row_idx = lax.iota(jnp.int32, 16) + (16 * lg)
                    col_idx = jnp.full((16,), dd, jnp.int32)
                    vec = plsc.load_gather(rows_v, [row_idx, col_idx])
                    tr_v[dd, pl.ds(16 * lg, 16)] = vec
                return c2

            lax.fori_loop(0, d, tr_body, 0)
            pltpu.sync_copy(tr_v, out_hbm.at[s, :, bt, :])
            return carry

        lax.fori_loop(0, units_per_w, unit_body, 0)

    return gather_kernel(table, ids_lin)


def _rotate_tc(x_lin, a3, block_s):
    n_s, d, two_bt, lanes = x_lin.shape
    bt = two_bt // 2

    def body(x_ref, a_ref, o_ref):
        v = x_ref[...]
        a = a_ref[...]
        asq = a * a
        recip = 1.0 / (1.0 + asq)
        c8 = (1.0 - asq) * recip
        s8 = (2.0 * a) * recip
        shape16 = (block_s, d, two_bt, lanes)
        c16 = jnp.broadcast_to(c8[:, :, :, None, :],
                               (block_s, d, bt, 2, lanes)).reshape(shape16)
        s16 = jnp.broadcast_to(s8[:, :, :, None, :],
                               (block_s, d, bt, 2, lanes)).reshape(shape16)
        i_idx = lax.broadcasted_iota(jnp.int32, shape16, 2)
        even = (i_idx % 2) == 0
        w = jnp.where(even, jnp.roll(v, -1, axis=2), jnp.roll(v, 1, axis=2))
        s_signed = jnp.where(even, -s16, s16)
        o_ref[...] = v * c16 + w * s_signed

    return pl.pallas_call(
        body,
        grid=(n_s // block_s,),
        in_specs=[
            pl.BlockSpec((block_s, d, two_bt, lanes), lambda i: (i, 0, 0, 0)),
            pl.BlockSpec((block_s, d, bt, lanes), lambda i: (i, 0, 0, 0)),
        ],
        out_specs=pl.BlockSpec((block_s, d, two_bt, lanes),
                               lambda i: (i, 0, 0, 0)),
        out_shape=jax.ShapeDtypeStruct((n_s, d, two_bt, lanes), jnp.float32),
        compiler_params=pltpu.CompilerParams(
            dimension_semantics=("arbitrary",),
        ),
    )(x_lin, a3)


def kernel(x, affix_ids, rotation_params):
    b, s, d, _ = x.shape
    nbt = b // _LANES
    x_lin = (x.transpose(1, 2, 3, 0)
              .reshape(s, d, 2, nbt, _LANES)
              .transpose(0, 1, 3, 2, 4)
              .reshape(s, d, 2 * nbt, _LANES))
    ids_lin = affix_ids.astype(jnp.int32).T.reshape(-1)
    a3 = _gather_transpose_sc(ids_lin, rotation_params)
    out_lin = _rotate_tc(x_lin, a3, block_s=4)
    out = (out_lin.reshape(s, d, nbt, 2, _LANES)
                  .transpose(2, 4, 0, 1, 3)
                  .reshape(b, s, d, 2))
    return out


# speedup vs baseline: 2.5404x; 1.1550x over previous
import functools

import jax
import jax.numpy as jnp
from jax import lax
from jax.experimental import pallas as pl
from jax.experimental.pallas import tpu as pltpu
from jax.experimental.pallas import tpu_sc as plsc

_LANES = 128
_NW = 32
_VOCAB_DIM = 64


def _gather_transpose_sc(ids_lin, table):
    (t_total,) = ids_lin.shape
    _, d = table.shape
    n_units = t_total // _LANES
    units_per_w = n_units // _NW
    mesh = plsc.VectorSubcoreMesh(core_axis_name="c", subcore_axis_name="s")

    @functools.partial(
        pl.kernel,
        mesh=mesh,
        out_type=jax.ShapeDtypeStruct((t_total // (8 * _LANES), d, 8, _LANES),
                                      jnp.float32),
        scratch_types=[
            pltpu.VMEM((units_per_w * _LANES,), jnp.int32),
            pltpu.VMEM((_LANES, d), jnp.float32),
            pltpu.VMEM((_LANES, d), jnp.float32),
            pltpu.VMEM((d, _LANES), jnp.float32),
            pltpu.VMEM((d, _LANES), jnp.float32),
            pltpu.SemaphoreType.DMA,
            pltpu.SemaphoreType.DMA,
            pltpu.SemaphoreType.DMA,
            pltpu.SemaphoreType.DMA,
        ],
        compiler_params=pltpu.CompilerParams(
            use_tc_tiling_on_sc=False,
            needs_layout_passes=False,
        ),
    )
    def gather_kernel(table_hbm, idx_hbm, out_hbm, idx_all,
                      rows_a, rows_b, tr_a, tr_b, sg_a, sg_b, sw_a, sw_b):
        wid = lax.axis_index("s") * 2 + lax.axis_index("c")
        base_u = wid * units_per_w

        pltpu.sync_copy(
            idx_hbm.at[pl.ds(pl.multiple_of(base_u * _LANES, _LANES),
                             units_per_w * _LANES)],
            idx_all)

        def start_gather(i, rows, sem):
            pltpu.async_copy(
                table_hbm.at[idx_all.at[pl.ds(i * _LANES, _LANES)]], rows, sem)

        def transpose(rows, tr):
            def tr_body(dd, c2):
                for lg in range(_LANES // 16):
                    row_idx = lax.iota(jnp.int32, 16) + (16 * lg)
                    col_idx = jnp.full((16,), dd, jnp.int32)
                    tr[dd, pl.ds(16 * lg, 16)] = plsc.load_gather(
                        rows, [row_idx, col_idx])
                return c2
            lax.fori_loop(0, d, tr_body, 0)

        def out_window(i):
            u = base_u + i
            return out_hbm.at[u // 8, :, u % 8, :]

        def start_write(i, tr, sem):
            pltpu.async_copy(tr, out_window(i), sem)

        def wait_gather(rows, sem):
            pltpu.make_async_copy(table_hbm.at[idx_all.at[pl.ds(0, _LANES)]],
                                  rows, sem).wait()

        def wait_write(i, tr, sem):
            pltpu.make_async_copy(tr, out_window(i), sem).wait()

        start_gather(0, rows_a, sg_a)

        def step(k, carry):
            i0 = 2 * k
            i1 = 2 * k + 1
            start_gather(i1, rows_b, sg_b)
            wait_gather(rows_a, sg_a)

            @pl.when(k > 0)
            def _():
                wait_write(i0, tr_a, sw_a)
            transpose(rows_a, tr_a)
            start_write(i0, tr_a, sw_a)

            @pl.when(k < (units_per_w // 2 - 1))
            def _():
                start_gather(i0 + 2, rows_a, sg_a)
            wait_gather(rows_b, sg_b)

            @pl.when(k > 0)
            def _():
                wait_write(i1, tr_b, sw_b)
            transpose(rows_b, tr_b)
            start_write(i1, tr_b, sw_b)
            return carry

        lax.fori_loop(0, units_per_w // 2, step, 0)
        wait_write(0, tr_a, sw_a)
        wait_write(0, tr_b, sw_b)

    return gather_kernel(table, ids_lin)


def _rotate_tc(x_lin, a3, block_s):
    n_s, d, two_bt, lanes = x_lin.shape
    bt = two_bt // 2

    def body(x_ref, a_ref, o_ref):
        v = x_ref[...]
        a = a_ref[...]
        asq = a * a
        recip = 1.0 / (1.0 + asq)
        c8 = (1.0 - asq) * recip
        s8 = (2.0 * a) * recip
        shape16 = (block_s, d, two_bt, lanes)
        c16 = jnp.broadcast_to(c8[:, :, :, None, :],
                               (block_s, d, bt, 2, lanes)).reshape(shape16)
        s16 = jnp.broadcast_to(s8[:, :, :, None, :],
                               (block_s, d, bt, 2, lanes)).reshape(shape16)
        i_idx = lax.broadcasted_iota(jnp.int32, shape16, 2)
        even = (i_idx % 2) == 0
        w = jnp.where(even, jnp.roll(v, -1, axis=2), jnp.roll(v, 1, axis=2))
        s_signed = jnp.where(even, -s16, s16)
        o_ref[...] = v * c16 + w * s_signed

    return pl.pallas_call(
        body,
        grid=(n_s // block_s,),
        in_specs=[
            pl.BlockSpec((block_s, d, two_bt, lanes), lambda i: (i, 0, 0, 0)),
            pl.BlockSpec((block_s, d, bt, lanes), lambda i: (i, 0, 0, 0)),
        ],
        out_specs=pl.BlockSpec((block_s, d, two_bt, lanes),
                               lambda i: (i, 0, 0, 0)),
        out_shape=jax.ShapeDtypeStruct((n_s, d, two_bt, lanes), jnp.float32),
        compiler_params=pltpu.CompilerParams(
            dimension_semantics=("arbitrary",),
        ),
    )(x_lin, a3)


def kernel(x, affix_ids, rotation_params):
    b, s, d, _ = x.shape
    nbt = b // _LANES
    x_lin = (x.transpose(1, 2, 3, 0)
              .reshape(s, d, 2, nbt, _LANES)
              .transpose(0, 1, 3, 2, 4)
              .reshape(s, d, 2 * nbt, _LANES))
    ids_lin = affix_ids.astype(jnp.int32).T.reshape(-1)
    a3 = _gather_transpose_sc(ids_lin, rotation_params)
    out_lin = _rotate_tc(x_lin, a3, block_s=4)
    out = (out_lin.reshape(s, d, nbt, 2, _LANES)
                  .transpose(2, 4, 0, 1, 3)
                  .reshape(b, s, d, 2))
    return out


# speedup vs baseline: 3.9611x; 1.5593x over previous
import functools

import jax
import jax.numpy as jnp
from jax import lax
from jax.experimental import pallas as pl
from jax.experimental.pallas import tpu as pltpu
from jax.experimental.pallas import tpu_sc as plsc

_LANES = 128
_NW = 32
_VOCAB_DIM = 64


def _gather_transpose_sc(ids_lin, table):
    (t_total,) = ids_lin.shape
    _, d = table.shape
    n_units = t_total // _LANES
    units_per_w = n_units // _NW
    mesh = plsc.VectorSubcoreMesh(core_axis_name="c", subcore_axis_name="s")

    @functools.partial(
        pl.kernel,
        mesh=mesh,
        out_type=jax.ShapeDtypeStruct((t_total // (8 * _LANES), d, 8, _LANES),
                                      jnp.float32),
        scratch_types=[
            pltpu.VMEM((units_per_w * _LANES,), jnp.int32),
            pltpu.VMEM((_LANES, d), jnp.float32),
            pltpu.VMEM((_LANES, d), jnp.float32),
            pltpu.VMEM((d, _LANES), jnp.float32),
            pltpu.VMEM((d, _LANES), jnp.float32),
            pltpu.SemaphoreType.DMA,
            pltpu.SemaphoreType.DMA,
            pltpu.SemaphoreType.DMA,
            pltpu.SemaphoreType.DMA,
        ],
        compiler_params=pltpu.CompilerParams(
            use_tc_tiling_on_sc=False,
            needs_layout_passes=False,
        ),
    )
    def gather_kernel(table_hbm, idx_hbm, out_hbm, idx_all,
                      rows_a, rows_b, tr_a, tr_b, sg_a, sg_b, sw_a, sw_b):
        wid = lax.axis_index("s") * 2 + lax.axis_index("c")
        base_u = wid * units_per_w

        pltpu.sync_copy(
            idx_hbm.at[pl.ds(pl.multiple_of(base_u * _LANES, _LANES),
                             units_per_w * _LANES)],
            idx_all)

        def start_gather(i, rows, sem):
            pltpu.async_copy(
                table_hbm.at[idx_all.at[pl.ds(i * _LANES, _LANES)]], rows, sem)

        iota16 = lax.iota(jnp.int32, 16)

        def transpose(rows, tr):
            def tr_body(c0, c2):
                dj = jnp.bitwise_and(c0 + iota16, d - 1)
                for lg in range(_LANES // 16):
                    lv = iota16 + (16 * lg)
                    vec = plsc.load_gather(rows, [lv, dj])
                    plsc.store_scatter(tr, [dj, lv], vec)
                return c2
            lax.fori_loop(0, d, tr_body, 0, unroll=4)

        def out_window(i):
            u = base_u + i
            return out_hbm.at[u // 8, :, u % 8, :]

        def start_write(i, tr, sem):
            pltpu.async_copy(tr, out_window(i), sem)

        def wait_gather(rows, sem):
            pltpu.make_async_copy(table_hbm.at[idx_all.at[pl.ds(0, _LANES)]],
                                  rows, sem).wait()

        def wait_write(i, tr, sem):
            pltpu.make_async_copy(tr, out_window(i), sem).wait()

        start_gather(0, rows_a, sg_a)

        def step(k, carry):
            i0 = 2 * k
            i1 = 2 * k + 1
            start_gather(i1, rows_b, sg_b)
            wait_gather(rows_a, sg_a)

            @pl.when(k > 0)
            def _():
                wait_write(i0, tr_a, sw_a)
            transpose(rows_a, tr_a)
            start_write(i0, tr_a, sw_a)

            @pl.when(k < (units_per_w // 2 - 1))
            def _():
                start_gather(i0 + 2, rows_a, sg_a)
            wait_gather(rows_b, sg_b)

            @pl.when(k > 0)
            def _():
                wait_write(i1, tr_b, sw_b)
            transpose(rows_b, tr_b)
            start_write(i1, tr_b, sw_b)
            return carry

        lax.fori_loop(0, units_per_w // 2, step, 0)
        wait_write(0, tr_a, sw_a)
        wait_write(0, tr_b, sw_b)

    return gather_kernel(table, ids_lin)


def _rotate_tc(x_lin, a3, block_s):
    n_s, d, two_bt, lanes = x_lin.shape
    bt = two_bt // 2

    def body(x_ref, a_ref, o_ref):
        v = x_ref[...]
        a = a_ref[...]
        asq = a * a
        recip = 1.0 / (1.0 + asq)
        c8 = (1.0 - asq) * recip
        s8 = (2.0 * a) * recip
        shape16 = (block_s, d, two_bt, lanes)
        c16 = jnp.broadcast_to(c8[:, :, :, None, :],
                               (block_s, d, bt, 2, lanes)).reshape(shape16)
        s16 = jnp.broadcast_to(s8[:, :, :, None, :],
                               (block_s, d, bt, 2, lanes)).reshape(shape16)
        i_idx = lax.broadcasted_iota(jnp.int32, shape16, 2)
        even = (i_idx % 2) == 0
        w = jnp.where(even, jnp.roll(v, -1, axis=2), jnp.roll(v, 1, axis=2))
        s_signed = jnp.where(even, -s16, s16)
        o_ref[...] = v * c16 + w * s_signed

    return pl.pallas_call(
        body,
        grid=(n_s // block_s,),
        in_specs=[
            pl.BlockSpec((block_s, d, two_bt, lanes), lambda i: (i, 0, 0, 0)),
            pl.BlockSpec((block_s, d, bt, lanes), lambda i: (i, 0, 0, 0)),
        ],
        out_specs=pl.BlockSpec((block_s, d, two_bt, lanes),
                               lambda i: (i, 0, 0, 0)),
        out_shape=jax.ShapeDtypeStruct((n_s, d, two_bt, lanes), jnp.float32),
        compiler_params=pltpu.CompilerParams(
            dimension_semantics=("arbitrary",),
        ),
    )(x_lin, a3)


def kernel(x, affix_ids, rotation_params):
    b, s, d, _ = x.shape
    nbt = b // _LANES
    x_lin = (x.transpose(1, 2, 3, 0)
              .reshape(s, d, 2, nbt, _LANES)
              .transpose(0, 1, 3, 2, 4)
              .reshape(s, d, 2 * nbt, _LANES))
    ids_lin = affix_ids.astype(jnp.int32).T.reshape(-1)
    a3 = _gather_transpose_sc(ids_lin, rotation_params)
    out_lin = _rotate_tc(x_lin, a3, block_s=4)
    out = (out_lin.reshape(s, d, nbt, 2, _LANES)
                  .transpose(2, 4, 0, 1, 3)
                  .reshape(b, s, d, 2))
    return out
